# Initial kernel scaffold; baseline (speedup 1.0000x reference)
import jax
import jax.numpy as jnp
from jax.experimental import pallas as pl


def kernel(x, w1, w2):
    raise NotImplementedError("write your pallas kernel here")


# speedup vs baseline: 1.1501x; 1.1501x over previous
import jax
import jax.numpy as jnp
from jax.experimental import pallas as pl
from jax.experimental.pallas import tpu as pltpu


def _make_se_kernel(hw_total):
    inv_hw = 1.0 / float(hw_total)

    def _body(x_ref, w1t_ref, w2_ref, o_ref):
        x = x_ref[...]
        pooled = jnp.sum(x, axis=-1, keepdims=True) * inv_hw
        h = jnp.sum(w1t_ref[...] * pooled, axis=0, keepdims=True)
        h = jnp.maximum(h, 0.0)
        s = jnp.sum(w2_ref[...] * h, axis=-1, keepdims=True)
        s = jax.nn.sigmoid(s)
        o_ref[...] = (x * s).astype(o_ref.dtype)

    return _body


def kernel(x, w1, w2):
    b, c, h, w = x.shape
    hw = h * w
    hidden = w1.shape[0]

    x_flat = x.reshape(b, c, hw).astype(jnp.float32)
    w1t = jnp.transpose(w1.astype(jnp.float32))
    w2f = w2.astype(jnp.float32)

    out = pl.pallas_call(
        _make_se_kernel(hw),
        out_shape=jax.ShapeDtypeStruct((b, c, hw), x.dtype),
        grid=(b,),
        in_specs=[
            pl.BlockSpec((None, c, hw), lambda i: (i, 0, 0)),
            pl.BlockSpec((c, hidden), lambda i: (0, 0)),
            pl.BlockSpec((c, hidden), lambda i: (0, 0)),
        ],
        out_specs=pl.BlockSpec((None, c, hw), lambda i: (i, 0, 0)),
        compiler_params=pltpu.CompilerParams(
            dimension_semantics=("parallel",),
            vmem_limit_bytes=100 * 1024 * 1024,
        ),
    )(x_flat, w1t, w2f)

    return out.reshape(b, c, h, w)
